# Initial kernel scaffold; baseline (speedup 1.0000x reference)
import jax
import jax.numpy as jnp
from jax.experimental import pallas as pl


def kernel(logits):
    raise NotImplementedError("write your pallas kernel here")


# speedup vs baseline: 11.8195x; 11.8195x over previous
import functools

import jax
import jax.numpy as jnp
from jax import lax
from jax.experimental import pallas as pl
from jax.experimental.pallas import tpu as pltpu
from jax.experimental.pallas import tpu_sc as plsc

_B = 128
_D = 32768
_K = 10
_NC = 2
_NS = 16
_NW = _NC * _NS
_RPW = _B // _NW
_L = 16
_NG = 8
_GI = _D // (_NG * _L)


def _bfly_max_f32(v, fsc, it16):
    for sh in (8, 4, 2, 1):
        fsc[...] = v
        w = plsc.load_gather(fsc, [jnp.bitwise_xor(it16, sh)])
        v = jnp.maximum(v, w)
    return v


def _bfly_max_i32(v, isc, it16):
    for sh in (8, 4, 2, 1):
        isc[...] = v
        w = plsc.load_gather(isc, [jnp.bitwise_xor(it16, sh)])
        v = jnp.maximum(v, w)
    return v


def _bfly_sum_i32(v, isc, it16):
    for sh in (8, 4, 2, 1):
        isc[...] = v
        w = plsc.load_gather(isc, [jnp.bitwise_xor(it16, sh)])
        v = v + w
    return v


def _row_threshold(row_v, fsc, isc, it16):
    minf = jnp.float32(-jnp.inf)

    cmaxes = []
    for g in range(_NG):
        def p1(i, m, g=g):
            return jnp.maximum(m, row_v[pl.ds((g * _GI + i) * _L, _L)])
        cmaxes.append(lax.fori_loop(0, _GI, p1,
                                    jnp.full((_L,), minf, jnp.float32)))

    def p2(_, carry):
        c, t, cm = carry
        m = cm[0]
        for g in range(1, _NG):
            m = jnp.maximum(m, cm[g])
        m = _bfly_max_f32(m, fsc, it16)
        best = jnp.full((_L,), -1, jnp.int32)
        for g in range(_NG):
            best = jnp.maximum(best, jnp.where(cm[g] == m, it16 + g * _L, -1))
        best = _bfly_max_i32(best, isc, it16)
        gsel = best >> 4
        lsel = best & 15

        def rs(u, cr):
            cnt, nm = cr
            idx = (gsel * _GI + u * _L + it16) * _L + lsel
            v = plsc.load_gather(row_v, [idx])
            cnt = cnt + jnp.where(v == m, 1, 0).astype(jnp.int32)
            nm = jnp.maximum(nm, jnp.where(v < m, v, minf))
            return cnt, nm

        cntv, nmv = lax.fori_loop(0, _GI // _L, rs,
                                  (jnp.zeros((_L,), jnp.int32),
                                   jnp.full((_L,), minf, jnp.float32)))
        t = jnp.where(c < _K, m, t)
        c = c + _bfly_sum_i32(cntv, isc, it16)
        nm = _bfly_max_f32(nmv, fsc, it16)
        new_cm = []
        for g in range(_NG):
            hit = jnp.logical_and(gsel == g, it16 == lsel)
            new_cm.append(jnp.where(hit, nm, cm[g]))
        return c, t, tuple(new_cm)

    _, t, _ = lax.fori_loop(0, _K, p2,
                            (jnp.zeros((_L,), jnp.int32),
                             jnp.full((_L,), minf, jnp.float32),
                             tuple(cmaxes)))
    return t


def _mask_row(row_v, tv):
    one = jnp.full((_L,), 1.0, jnp.float32)
    zero = jnp.zeros((_L,), jnp.float32)

    def p3(i, carry):
        v = row_v[pl.ds(i * _L, _L)]
        row_v[pl.ds(i * _L, _L)] = jnp.where(v >= tv, one, zero)
        return carry

    lax.fori_loop(0, _D // _L, p3, jnp.int32(0))


_mesh = plsc.VectorSubcoreMesh(core_axis_name="c", subcore_axis_name="s",
                               num_cores=_NC, num_subcores=_NS)


@functools.partial(
    pl.kernel,
    out_type=jax.ShapeDtypeStruct((_B, _D), jnp.float32),
    mesh=_mesh,
    scratch_types=[pltpu.VMEM((_D,), jnp.float32),
                   pltpu.VMEM((_L,), jnp.float32),
                   pltpu.VMEM((_L,), jnp.int32)],
    compiler_params=pltpu.CompilerParams(needs_layout_passes=False),
)
def _topk_mask_kernel(x_hbm, out_hbm, row_v, fsc, isc):
    wid = lax.axis_index("s") * _NC + lax.axis_index("c")
    it16 = lax.iota(jnp.int32, _L)
    for i in range(_RPW):
        r = wid * _RPW + i
        pltpu.sync_copy(x_hbm.at[r], row_v)
        tv = _row_threshold(row_v, fsc, isc, it16)
        _mask_row(row_v, tv)
        pltpu.sync_copy(row_v, out_hbm.at[r])


def kernel(logits):
    x = logits.reshape(_B, _D)
    out = _topk_mask_kernel(x)
    return out[..., None]


# speedup vs baseline: 40.2249x; 3.4033x over previous
import functools

import jax
import jax.numpy as jnp
from jax import lax
from jax.experimental import pallas as pl
from jax.experimental.pallas import tpu as pltpu
from jax.experimental.pallas import tpu_sc as plsc

_B = 128
_D = 32768
_K = 10
_NC = 2
_NS = 16
_NW = _NC * _NS
_RPW = _B // _NW
_L = 16
_NG = 8
_GI = _D // (_NG * _L)
_NBUF = 3


def _bfly_max_f32(v, fsc, it16):
    for sh in (8, 4, 2, 1):
        fsc[...] = v
        w = plsc.load_gather(fsc, [jnp.bitwise_xor(it16, sh)])
        v = jnp.maximum(v, w)
    return v


def _bfly_max_i32(v, isc, it16):
    for sh in (8, 4, 2, 1):
        isc[...] = v
        w = plsc.load_gather(isc, [jnp.bitwise_xor(it16, sh)])
        v = jnp.maximum(v, w)
    return v


def _bfly_sum_i32(v, isc, it16):
    for sh in (8, 4, 2, 1):
        isc[...] = v
        w = plsc.load_gather(isc, [jnp.bitwise_xor(it16, sh)])
        v = v + w
    return v


def _row_threshold(row_v, fsc, isc, it16):
    minf = jnp.float32(-jnp.inf)

    acc0 = tuple(jnp.full((_L,), minf, jnp.float32) for _ in range(_NG))

    @plsc.parallel_loop(0, _GI, carry=acc0, unroll=4)
    def p1(i, acc):
        return tuple(
            jnp.maximum(acc[g], row_v[pl.ds((g * _GI + i) * _L, _L)])
            for g in range(_NG)
        )

    cmaxes = p1

    def p2(_, carry):
        c, t, cm = carry
        m = cm[0]
        for g in range(1, _NG):
            m = jnp.maximum(m, cm[g])
        m = _bfly_max_f32(m, fsc, it16)
        best = jnp.full((_L,), -1, jnp.int32)
        for g in range(_NG):
            best = jnp.maximum(best, jnp.where(cm[g] == m, it16 + g * _L, -1))
        best = _bfly_max_i32(best, isc, it16)
        gsel = best >> 4
        lsel = best & 15
        base = (gsel * _GI + it16) * _L + lsel

        def rs(u, cr):
            cnt, nm = cr
            v = plsc.load_gather(row_v, [base + u * (_L * _L)])
            cnt = cnt + jnp.where(v == m, 1, 0).astype(jnp.int32)
            nm = jnp.maximum(nm, jnp.where(v < m, v, minf))
            return cnt, nm

        cntv, nmv = lax.fori_loop(0, _GI // _L, rs,
                                  (jnp.zeros((_L,), jnp.int32),
                                   jnp.full((_L,), minf, jnp.float32)))
        t = jnp.where(c < _K, m, t)
        c = c + _bfly_sum_i32(cntv, isc, it16)
        nm = _bfly_max_f32(nmv, fsc, it16)
        new_cm = []
        for g in range(_NG):
            hit = jnp.logical_and(gsel == g, it16 == lsel)
            new_cm.append(jnp.where(hit, nm, cm[g]))
        return c, t, tuple(new_cm)

    _, t, _ = lax.fori_loop(0, _K, p2,
                            (jnp.zeros((_L,), jnp.int32),
                             jnp.full((_L,), minf, jnp.float32),
                             tuple(cmaxes)))
    return t


def _mask_row(row_v, tv):
    one = jnp.full((_L,), 1.0, jnp.float32)
    zero = jnp.zeros((_L,), jnp.float32)

    @plsc.parallel_loop(0, _D // _L, unroll=8)
    def p3(i):
        v = row_v[pl.ds(i * _L, _L)]
        row_v[pl.ds(i * _L, _L)] = jnp.where(v >= tv, one, zero)


_mesh = plsc.VectorSubcoreMesh(core_axis_name="c", subcore_axis_name="s",
                               num_cores=_NC, num_subcores=_NS)


@functools.partial(
    pl.kernel,
    out_type=jax.ShapeDtypeStruct((_B, _D), jnp.float32),
    mesh=_mesh,
    scratch_types=(
        [pltpu.VMEM((_D,), jnp.float32) for _ in range(_NBUF)]
        + [pltpu.VMEM((_L,), jnp.float32),
           pltpu.VMEM((_L,), jnp.int32)]
        + [pltpu.SemaphoreType.DMA] * (2 * _NBUF)
    ),
    compiler_params=pltpu.CompilerParams(needs_layout_passes=False, use_tc_tiling_on_sc=False),
)
def _topk_mask_kernel(x_hbm, out_hbm, *scratch):
    rows_v = scratch[:_NBUF]
    fsc, isc = scratch[_NBUF], scratch[_NBUF + 1]
    sems = scratch[_NBUF + 2:]
    in_sems = sems[:_NBUF]
    out_sems = sems[_NBUF:]
    wid = lax.axis_index("s") * _NC + lax.axis_index("c")
    it16 = lax.iota(jnp.int32, _L)
    r0 = wid * _RPW

    in_copies = [None] * _RPW
    out_copies = [None] * _RPW
    in_copies[0] = pltpu.async_copy(x_hbm.at[r0], rows_v[0], in_sems[0])
    for i in range(_RPW):
        p = i % _NBUF
        if i + 1 < _RPW:
            q = (i + 1) % _NBUF
            if i + 1 >= _NBUF:
                out_copies[i + 1 - _NBUF].wait()
            in_copies[i + 1] = pltpu.async_copy(
                x_hbm.at[r0 + i + 1], rows_v[q], in_sems[q])
        in_copies[i].wait()
        row_v = rows_v[p]
        tv = _row_threshold(row_v, fsc, isc, it16)
        _mask_row(row_v, tv)
        out_copies[i] = pltpu.async_copy(row_v, out_hbm.at[r0 + i],
                                         out_sems[p])
    for i in range(max(0, _RPW - _NBUF), _RPW):
        out_copies[i].wait()


def kernel(logits):
    x = logits.reshape(_B, _D)
    out = _topk_mask_kernel(x)
    return out[..., None]


# speedup vs baseline: 42.6290x; 1.0598x over previous
import functools

import jax
import jax.numpy as jnp
from jax import lax
from jax.experimental import pallas as pl
from jax.experimental.pallas import tpu as pltpu
from jax.experimental.pallas import tpu_sc as plsc

_B = 128
_D = 32768
_K = 10
_NC = 2
_NS = 16
_NW = _NC * _NS
_RPW = _B // _NW
_L = 16
_NG = 8
_GI = _D // (_NG * _L)
_NBUF = 3


def _bfly_max_f32(v, fsc, it16):
    for sh in (8, 4, 2, 1):
        fsc[...] = v
        w = plsc.load_gather(fsc, [jnp.bitwise_xor(it16, sh)])
        v = jnp.maximum(v, w)
    return v


def _bfly_max_i32(v, isc, it16):
    for sh in (8, 4, 2, 1):
        isc[...] = v
        w = plsc.load_gather(isc, [jnp.bitwise_xor(it16, sh)])
        v = jnp.maximum(v, w)
    return v


def _bfly_sum_i32(v, isc, it16):
    for sh in (8, 4, 2, 1):
        isc[...] = v
        w = plsc.load_gather(isc, [jnp.bitwise_xor(it16, sh)])
        v = v + w
    return v


def _row_threshold(row_v, fsc, isc, it16):
    minf = jnp.float32(-jnp.inf)

    acc0 = tuple(jnp.full((_L,), minf, jnp.float32) for _ in range(_NG))

    @plsc.parallel_loop(0, _GI, carry=acc0, unroll=8)
    def p1(i, acc):
        return tuple(
            jnp.maximum(acc[g], row_v[pl.ds((g * _GI + i) * _L, _L)])
            for g in range(_NG)
        )

    cmaxes = p1

    def p2(_, carry):
        c, t, cm = carry
        m = cm[0]
        best = it16
        for g in range(1, _NG):
            sel = cm[g] > m
            m = jnp.maximum(m, cm[g])
            best = jnp.where(sel, it16 + g * _L, best)
        for sh in (8, 4, 2, 1):
            fsc[...] = m
            isc[...] = best
            perm = jnp.bitwise_xor(it16, sh)
            wm = plsc.load_gather(fsc, [perm])
            wb = plsc.load_gather(isc, [perm])
            sel = wm > m
            m = jnp.maximum(m, wm)
            best = jnp.where(sel, wb, best)
        gsel = best >> 4
        lsel = best & 15
        base = (gsel * _GI + it16) * _L + lsel

        cnt0 = jnp.zeros((_L,), jnp.int32)
        cnt1 = jnp.zeros((_L,), jnp.int32)
        nm0 = jnp.full((_L,), minf, jnp.float32)
        nm1 = jnp.full((_L,), minf, jnp.float32)
        for u in range(_GI // _L):
            v = plsc.load_gather(row_v, [base + u * (_L * _L)])
            if u % 2 == 0:
                cnt0 = cnt0 + jnp.where(v == m, 1, 0).astype(jnp.int32)
                nm0 = jnp.maximum(nm0, jnp.where(v < m, v, minf))
            else:
                cnt1 = cnt1 + jnp.where(v == m, 1, 0).astype(jnp.int32)
                nm1 = jnp.maximum(nm1, jnp.where(v < m, v, minf))
        t = jnp.where(c < _K, m, t)
        c = c + _bfly_sum_i32(cnt0 + cnt1, isc, it16)
        nm = _bfly_max_f32(jnp.maximum(nm0, nm1), fsc, it16)
        new_cm = []
        for g in range(_NG):
            hit = jnp.logical_and(gsel == g, it16 == lsel)
            new_cm.append(jnp.where(hit, nm, cm[g]))
        return c, t, tuple(new_cm)

    _, t, _ = lax.fori_loop(0, _K, p2,
                            (jnp.zeros((_L,), jnp.int32),
                             jnp.full((_L,), minf, jnp.float32),
                             tuple(cmaxes)))
    return t


def _mask_row(row_v, tv):
    one = jnp.full((_L,), 1.0, jnp.float32)
    zero = jnp.zeros((_L,), jnp.float32)

    @plsc.parallel_loop(0, _D // _L, unroll=16)
    def p3(i):
        v = row_v[pl.ds(i * _L, _L)]
        row_v[pl.ds(i * _L, _L)] = jnp.where(v >= tv, one, zero)


_mesh = plsc.VectorSubcoreMesh(core_axis_name="c", subcore_axis_name="s",
                               num_cores=_NC, num_subcores=_NS)


@functools.partial(
    pl.kernel,
    out_type=jax.ShapeDtypeStruct((_B, _D), jnp.float32),
    mesh=_mesh,
    scratch_types=(
        [pltpu.VMEM((_D,), jnp.float32) for _ in range(_NBUF)]
        + [pltpu.VMEM((_L,), jnp.float32),
           pltpu.VMEM((_L,), jnp.int32)]
        + [pltpu.SemaphoreType.DMA] * (2 * _NBUF)
    ),
    compiler_params=pltpu.CompilerParams(needs_layout_passes=False, use_tc_tiling_on_sc=False),
)
def _topk_mask_kernel(x_hbm, out_hbm, *scratch):
    rows_v = scratch[:_NBUF]
    fsc, isc = scratch[_NBUF], scratch[_NBUF + 1]
    sems = scratch[_NBUF + 2:]
    in_sems = sems[:_NBUF]
    out_sems = sems[_NBUF:]
    wid = lax.axis_index("s") * _NC + lax.axis_index("c")
    it16 = lax.iota(jnp.int32, _L)
    r0 = wid * _RPW

    in_copies = [None] * _RPW
    out_copies = [None] * _RPW
    in_copies[0] = pltpu.async_copy(x_hbm.at[r0], rows_v[0], in_sems[0])
    for i in range(_RPW):
        p = i % _NBUF
        if i + 1 < _RPW:
            q = (i + 1) % _NBUF
            if i + 1 >= _NBUF:
                out_copies[i + 1 - _NBUF].wait()
            in_copies[i + 1] = pltpu.async_copy(
                x_hbm.at[r0 + i + 1], rows_v[q], in_sems[q])
        in_copies[i].wait()
        row_v = rows_v[p]
        tv = _row_threshold(row_v, fsc, isc, it16)
        _mask_row(row_v, tv)
        out_copies[i] = pltpu.async_copy(row_v, out_hbm.at[r0 + i],
                                         out_sems[p])
    for i in range(max(0, _RPW - _NBUF), _RPW):
        out_copies[i].wait()


def kernel(logits):
    x = logits.reshape(_B, _D)
    out = _topk_mask_kernel(x)
    return out[..., None]
